# baseline (speedup 1.0000x reference)
import dataclasses
import functools

import jax
import jax.numpy as jnp
from jax import lax
from jax.experimental import pallas as pl
from jax.experimental.pallas import tpu as pltpu
from jax.experimental.pallas import tpu_sc as plsc

NX = 512
NY = 512
NZ = 1
C = 64
P = 32768
NSLOT = NX * NY * NZ
NCORES = 2
NSUB = 16
NW = NCORES * NSUB
SLOTS_PER_W = NSLOT // NW
L = 16
NZROWS = 128
ROWS_PER_DMA = 128
SENT = 0x7FFFFFFF


def _sc_scatter(idx, feat_ext):
    mesh = plsc.VectorSubcoreMesh(core_axis_name="c", subcore_axis_name="s")
    cp = pltpu.CompilerParams()
    if "needs_layout_passes" in pltpu.CompilerParams.__dataclass_fields__:
        cp = dataclasses.replace(cp, needs_layout_passes=False)

    @functools.partial(
        pl.kernel,
        mesh=mesh,
        compiler_params=cp,
        out_type=jax.ShapeDtypeStruct((NSLOT, 128), jnp.float32),
        scratch_types=[
            pltpu.VMEM((P,), jnp.int32),
            pltpu.VMEM((SLOTS_PER_W,), jnp.int32),
            pltpu.VMEM((L,), jnp.int32),
            pltpu.VMEM((ROWS_PER_DMA, 128), jnp.float32),
            pltpu.SemaphoreType.DMA,
        ],
    )
    def body(idx_hbm, feat_hbm, t_hbm, idx_v, src_v, key_v, rows_v, sem):
        wid = lax.axis_index("c") * NSUB + lax.axis_index("s")
        base = wid * SLOTS_PER_W
        lanes = lax.iota(jnp.int32, L)

        pltpu.sync_copy(idx_hbm, idx_v)

        @pl.loop(0, SLOTS_PER_W, step=L)
        def _(j):
            src_v[pl.ds(j, L)] = P + ((j + lanes + wid * 8) & (NZROWS - 1))

        @pl.loop(0, P, step=L)
        def _(p0):
            v_idx = idx_v[pl.ds(p0, L)]
            local = v_idx - base
            inr = (local >= 0) & (local < SLOTS_PER_W)
            pid = p0 + lanes
            combined = jnp.where(inr, local * P + pid, SENT)
            s = lax.sort(combined)
            key_v[...] = s
            nxt = plsc.load_gather(key_v, [jnp.minimum(lanes + 1, L - 1)])
            run_end = (s >> 15) != (nxt >> 15)
            winner = (s != SENT) & (run_end | (lanes == L - 1))
            plsc.store_scatter(src_v, [s >> 15], s & (P - 1), mask=winner)

        @pl.loop(0, SLOTS_PER_W, step=ROWS_PER_DMA)
        def _(j):
            gat = pltpu.async_copy(
                feat_hbm.at[src_v.at[pl.ds(j, ROWS_PER_DMA)]], rows_v, sem)
            gat.wait()
            pltpu.sync_copy(rows_v, t_hbm.at[pl.ds(base + j, ROWS_PER_DMA)])

    return body(idx, feat_ext)


def _tc_transpose(t):
    blk = 4096

    def body(t_ref, o_ref):
        o_ref[...] = t_ref[...][:, :C].T

    return pl.pallas_call(
        body,
        grid=(NSLOT // blk,),
        in_specs=[pl.BlockSpec((blk, 128), lambda i: (i, 0))],
        out_specs=pl.BlockSpec((C, blk), lambda i: (0, i)),
        out_shape=jax.ShapeDtypeStruct((C, NSLOT), jnp.float32),
    )(t)


def kernel(pillar_features, coords):
    idx = (coords[:, 0] + coords[:, 1] * NX + coords[:, 2]).astype(jnp.int32)
    feat_ext = jnp.zeros((P + NZROWS, 128), jnp.float32)
    feat_ext = lax.dynamic_update_slice(feat_ext, pillar_features, (0, 0))
    t = _sc_scatter(idx, feat_ext)
    out = _tc_transpose(t)
    return out.reshape(1, C * NZ, NY, NX)


# speedup vs baseline: 1.6322x; 1.6322x over previous
import dataclasses
import functools

import jax
import jax.numpy as jnp
from jax import lax
from jax.experimental import pallas as pl
from jax.experimental.pallas import tpu as pltpu
from jax.experimental.pallas import tpu_sc as plsc

NX = 512
NY = 512
NZ = 1
C = 64
P = 32768
NSLOT = NX * NY * NZ
NCORES = 2
NSUB = 16
NW = NCORES * NSUB
SLOTS_PER_W = NSLOT // NW
L = 16
NZROWS = 2048
ROWS_PER_DMA = 128
SENT = 0x7FFFFFFF


def _tc_build_table(feat_t):
    blk = 2048
    nblk = P // blk

    def body(x_ref, o_ref):
        i = pl.program_id(0)

        @pl.when(i < nblk)
        def _():
            o_ref[:, :C] = x_ref[...].T
            o_ref[:, C:] = jnp.zeros((blk, 128 - C), jnp.float32)

        @pl.when(i >= nblk)
        def _():
            o_ref[...] = jnp.zeros((blk, 128), jnp.float32)

    return pl.pallas_call(
        body,
        grid=(nblk + NZROWS // blk,),
        in_specs=[pl.BlockSpec((C, blk), lambda i: (0, jnp.minimum(i, nblk - 1)))],
        out_specs=pl.BlockSpec((blk, 128), lambda i: (i, 0)),
        out_shape=jax.ShapeDtypeStruct((P + NZROWS, 128), jnp.float32),
    )(feat_t)


def _sc_scatter(idx, feat_ext):
    mesh = plsc.VectorSubcoreMesh(core_axis_name="c", subcore_axis_name="s")
    cp = pltpu.CompilerParams()
    if "needs_layout_passes" in pltpu.CompilerParams.__dataclass_fields__:
        cp = dataclasses.replace(cp, needs_layout_passes=False)

    @functools.partial(
        pl.kernel,
        mesh=mesh,
        compiler_params=cp,
        out_type=jax.ShapeDtypeStruct((NSLOT, 128), jnp.float32),
        scratch_types=[
            pltpu.VMEM((P,), jnp.int32),
            pltpu.VMEM((SLOTS_PER_W,), jnp.int32),
            pltpu.VMEM((L,), jnp.int32),
            pltpu.VMEM((ROWS_PER_DMA, 128), jnp.float32),
            pltpu.VMEM((ROWS_PER_DMA, 128), jnp.float32),
            pltpu.SemaphoreType.DMA,
            pltpu.SemaphoreType.DMA,
        ],
    )
    def body(idx_hbm, feat_hbm, t_hbm, idx_v, src_v, key_v, rows_a, rows_b,
             sem_a, sem_b):
        wid = lax.axis_index("c") * NSUB + lax.axis_index("s")
        base = wid * SLOTS_PER_W
        lanes = lax.iota(jnp.int32, L)

        pltpu.sync_copy(idx_hbm, idx_v)

        @pl.loop(0, SLOTS_PER_W, step=L)
        def _(j):
            src_v[pl.ds(j, L)] = P + ((j + lanes + wid * 64) & (NZROWS - 1))

        @pl.loop(0, P, step=L)
        def _(p0):
            v_idx = idx_v[pl.ds(p0, L)]
            local = v_idx - base
            inr = local.astype(jnp.uint32) < jnp.uint32(SLOTS_PER_W)

            @pl.when(jnp.any(inr))
            def _():
                pid = p0 + lanes
                combined = jnp.where(inr, local * P + pid, SENT)
                s = lax.sort(combined)
                key_v[...] = s
                nxt = plsc.load_gather(key_v, [jnp.minimum(lanes + 1, L - 1)])
                run_end = (s >> 15) != (nxt >> 15)
                winner = (s != SENT) & (run_end | (lanes == L - 1))
                plsc.store_scatter(src_v, [s >> 15], s & (P - 1), mask=winner)

        @pl.loop(0, SLOTS_PER_W, step=2 * ROWS_PER_DMA)
        def _(j):
            ga = pltpu.async_copy(
                feat_hbm.at[src_v.at[pl.ds(j, ROWS_PER_DMA)]], rows_a, sem_a)
            gb = pltpu.async_copy(
                feat_hbm.at[src_v.at[pl.ds(j + ROWS_PER_DMA, ROWS_PER_DMA)]],
                rows_b, sem_b)
            ga.wait()
            wa = pltpu.async_copy(
                rows_a, t_hbm.at[pl.ds(base + j, ROWS_PER_DMA)], sem_a)
            gb.wait()
            wb = pltpu.async_copy(
                rows_b, t_hbm.at[pl.ds(base + j + ROWS_PER_DMA, ROWS_PER_DMA)],
                sem_b)
            wa.wait()
            wb.wait()

    return body(idx, feat_ext)


def _tc_transpose(t):
    blk = 4096
    yb = blk // NX

    def body(t_ref, o_ref):
        x = t_ref[...]
        for y in range(yb):
            o_ref[:, y, :] = x[y * NX:(y + 1) * NX, :C].T

    return pl.pallas_call(
        body,
        grid=(NSLOT // blk,),
        in_specs=[pl.BlockSpec((blk, 128), lambda i: (i, 0))],
        out_specs=pl.BlockSpec((C, yb, NX), lambda i: (0, i, 0)),
        out_shape=jax.ShapeDtypeStruct((C, NY, NX), jnp.float32),
    )(t)


def kernel(pillar_features, coords):
    idx = (coords[:, 0] + coords[:, 1] * NX + coords[:, 2]).astype(jnp.int32)
    feat_ext = _tc_build_table(pillar_features.T)
    t = _sc_scatter(idx, feat_ext)
    out = _tc_transpose(t)
    return out.reshape(1, C * NZ, NY, NX)


# speedup vs baseline: 1.7507x; 1.0726x over previous
import dataclasses
import functools

import jax
import jax.numpy as jnp
from jax import lax
from jax.experimental import pallas as pl
from jax.experimental.pallas import tpu as pltpu
from jax.experimental.pallas import tpu_sc as plsc

NX = 512
NY = 512
NZ = 1
C = 64
P = 32768
NSLOT = NX * NY * NZ
NCORES = 2
NSUB = 16
NW = NCORES * NSUB
SLOTS_PER_W = NSLOT // NW
L = 16
RPC = 128
NTRASH = NW * RPC
SENT = 0x7FFFFFFF


def _tc_build_table(feat_t):
    blk = 2048

    def body(x_ref, o_ref):
        o_ref[:, :C] = x_ref[...].T
        o_ref[:, C:] = jnp.zeros((blk, 128 - C), jnp.float32)

    return pl.pallas_call(
        body,
        grid=(P // blk,),
        in_specs=[pl.BlockSpec((C, blk), lambda i: (0, i))],
        out_specs=pl.BlockSpec((blk, 128), lambda i: (i, 0)),
        out_shape=jax.ShapeDtypeStruct((P, 128), jnp.float32),
    )(feat_t)


def _sc_scatter(idx, feat_ext):
    mesh = plsc.VectorSubcoreMesh(core_axis_name="c", subcore_axis_name="s")
    cp = pltpu.CompilerParams()
    if "needs_layout_passes" in pltpu.CompilerParams.__dataclass_fields__:
        cp = dataclasses.replace(cp, needs_layout_passes=False)

    nvec = SLOTS_PER_W // L
    cap = SLOTS_PER_W + RPC

    @functools.partial(
        pl.kernel,
        mesh=mesh,
        compiler_params=cp,
        out_type=(
            jax.ShapeDtypeStruct((NSLOT + NTRASH, 128), jnp.float32),
            jax.ShapeDtypeStruct((NSLOT,), jnp.int32),
        ),
        scratch_types=[
            pltpu.VMEM((P,), jnp.int32),
            pltpu.VMEM((SLOTS_PER_W,), jnp.int32),
            pltpu.VMEM((L,), jnp.int32),
            pltpu.VMEM((cap,), jnp.int32),
            pltpu.VMEM((cap,), jnp.int32),
            pltpu.VMEM((cap // RPC, RPC), jnp.int32),
            pltpu.VMEM((RPC, 128), jnp.float32),
            pltpu.SemaphoreType.DMA,
        ],
    )
    def body(idx_hbm, feat_hbm, t_hbm, src_hbm, idx_v, src_v, key_v,
             pid_v, dst_v, dst2_v, rows_v, sem):
        wid = lax.axis_index("c") * NSUB + lax.axis_index("s")
        base = wid * SLOTS_PER_W
        lanes = lax.iota(jnp.int32, L)

        pltpu.sync_copy(idx_hbm, idx_v)

        @pl.loop(0, SLOTS_PER_W, step=L)
        def _(j):
            src_v[pl.ds(j, L)] = jnp.full((L,), -1, jnp.int32)

        @pl.loop(0, P, step=L)
        def _(p0):
            v_idx = idx_v[pl.ds(p0, L)]
            local = v_idx - base
            inr = local.astype(jnp.uint32) < jnp.uint32(SLOTS_PER_W)

            @pl.when(jnp.any(inr))
            def _():
                pid = p0 + lanes
                combined = jnp.where(inr, local * P + pid, SENT)
                s = lax.sort(combined)
                key_v[...] = s
                nxt = plsc.load_gather(key_v, [jnp.minimum(lanes + 1, L - 1)])
                run_end = (s >> 15) != (nxt >> 15)
                winner = (s != SENT) & (run_end | (lanes == L - 1))
                plsc.store_scatter(src_v, [s >> 15], s & (P - 1), mask=winner)

        pltpu.sync_copy(src_v, src_hbm.at[pl.ds(base, SLOTS_PER_W)])

        def compact(k, off):
            v = src_v[pl.ds(k * L, L)]
            m = v >= 0
            plsc.store_compressed(pid_v.at[pl.ds(off, L)], v, mask=m)
            plsc.store_compressed(dst_v.at[pl.ds(off, L)],
                                  base + k * L + lanes, mask=m)
            return off + jnp.sum(m.astype(jnp.int32))

        nwin = lax.fori_loop(0, nvec, compact, jnp.int32(0))

        trash = NSLOT + wid * RPC + lanes

        @pl.loop(0, RPC, step=L)
        def _(t):
            pid_v[pl.ds(nwin + t, L)] = jnp.zeros((L,), jnp.int32)
            dst_v[pl.ds(nwin + t, L)] = trash + t

        @pl.loop(0, cap // L)
        def _(i):
            dst2_v[i >> 3, pl.ds((i & 7) * L, L)] = dst_v[pl.ds(i * L, L)]

        def move(k, carry):
            gat = pltpu.async_copy(
                feat_hbm.at[pid_v.at[pl.ds(k * RPC, RPC)]], rows_v, sem)
            gat.wait()
            sc = pltpu.async_copy(rows_v, t_hbm.at[dst2_v.at[k]], sem)
            sc.wait()
            return carry

        lax.fori_loop(0, (nwin + RPC - 1) // RPC, move, jnp.int32(0))

    return body(idx, feat_ext)


def _tc_transpose(t, src):
    blk = 4096
    yb = blk // NX

    def body(t_ref, s_ref, o_ref):
        x = t_ref[...]
        s = s_ref[0, 0, :]
        for y in range(yb):
            m = (s[y * NX:(y + 1) * NX] >= 0)[None, :]
            o_ref[:, y, :] = jnp.where(m, x[y * NX:(y + 1) * NX, :C].T, 0.0)

    return pl.pallas_call(
        body,
        grid=(NSLOT // blk,),
        in_specs=[
            pl.BlockSpec((blk, 128), lambda i: (i, 0)),
            pl.BlockSpec((1, 1, blk), lambda i: (i, 0, 0)),
        ],
        out_specs=pl.BlockSpec((C, yb, NX), lambda i: (0, i, 0)),
        out_shape=jax.ShapeDtypeStruct((C, NY, NX), jnp.float32),
    )(t, src.reshape(NSLOT // blk, 1, blk))


def kernel(pillar_features, coords):
    idx = (coords[:, 0] + coords[:, 1] * NX + coords[:, 2]).astype(jnp.int32)
    feat_ext = _tc_build_table(pillar_features.T)
    t, src = _sc_scatter(idx, feat_ext)
    out = _tc_transpose(t, src)
    return out.reshape(1, C * NZ, NY, NX)


# speedup vs baseline: 2.0728x; 1.1840x over previous
import dataclasses
import functools

import jax
import jax.numpy as jnp
from jax import lax
from jax.experimental import pallas as pl
from jax.experimental.pallas import tpu as pltpu
from jax.experimental.pallas import tpu_sc as plsc

NX = 512
NY = 512
NZ = 1
C = 64
P = 32768
NSLOT = NX * NY * NZ
NCORES = 2
NSUB = 16
NW = NCORES * NSUB
SLOTS_PER_W = NSLOT // NW
L = 16
RPC = 128
NTRASH = NW * RPC
SENT = 0x7FFFFFFF


def _tc_build_table(feat_t):
    blk = 2048

    def body(x_ref, o_ref):
        o_ref[:, :C] = x_ref[...].T
        o_ref[:, C:] = jnp.zeros((blk, 128 - C), jnp.float32)

    return pl.pallas_call(
        body,
        grid=(P // blk,),
        in_specs=[pl.BlockSpec((C, blk), lambda i: (0, i))],
        out_specs=pl.BlockSpec((blk, 128), lambda i: (i, 0)),
        out_shape=jax.ShapeDtypeStruct((P, 128), jnp.float32),
    )(feat_t)


def _sc_scatter(idx, feat_ext):
    mesh = plsc.VectorSubcoreMesh(core_axis_name="c", subcore_axis_name="s")
    cp = pltpu.CompilerParams()
    if "needs_layout_passes" in pltpu.CompilerParams.__dataclass_fields__:
        cp = dataclasses.replace(cp, needs_layout_passes=False)

    nvec = SLOTS_PER_W // L
    cap = SLOTS_PER_W + RPC

    @functools.partial(
        pl.kernel,
        mesh=mesh,
        compiler_params=cp,
        out_type=(
            jax.ShapeDtypeStruct((NSLOT + NTRASH, 128), jnp.float32),
            jax.ShapeDtypeStruct((NSLOT,), jnp.int32),
        ),
        scratch_types=[
            pltpu.VMEM((P,), jnp.int32),
            pltpu.VMEM((SLOTS_PER_W,), jnp.int32),
            pltpu.VMEM((cap,), jnp.int32),
            pltpu.VMEM((cap,), jnp.int32),
            pltpu.VMEM((cap // RPC, RPC), jnp.int32),
            pltpu.VMEM((RPC, 128), jnp.float32),
            pltpu.SemaphoreType.DMA,
        ],
    )
    def body(idx_hbm, feat_hbm, t_hbm, src_hbm, idx_v, src_v,
             pid_v, dst_v, dst2_v, rows_v, sem):
        wid = lax.axis_index("c") * NSUB + lax.axis_index("s")
        base = wid * SLOTS_PER_W
        lanes = lax.iota(jnp.int32, L)

        pltpu.sync_copy(idx_hbm, idx_v)

        @pl.loop(0, SLOTS_PER_W, step=L)
        def _(j):
            src_v[pl.ds(j, L)] = jnp.full((L,), -1, jnp.int32)

        @pl.loop(0, P, step=L)
        def _(p0):
            v_idx = idx_v[pl.ds(p0, L)]
            local = v_idx - base
            inr = local.astype(jnp.uint32) < jnp.uint32(SLOTS_PER_W)
            _, last = plsc.scan_count(local, mask=inr)
            plsc.store_scatter(src_v, [local], p0 + lanes, mask=last & inr)

        pltpu.sync_copy(src_v, src_hbm.at[pl.ds(base, SLOTS_PER_W)])

        def compact(k, off):
            v = src_v[pl.ds(k * L, L)]
            m = v >= 0
            plsc.store_compressed(pid_v.at[pl.ds(off, L)], v, mask=m)
            plsc.store_compressed(dst_v.at[pl.ds(off, L)],
                                  base + k * L + lanes, mask=m)
            return off + plsc.all_reduce_population_count(m)[0]

        nwin = lax.fori_loop(0, nvec, compact, jnp.int32(0))

        trash = NSLOT + wid * RPC + lanes

        @pl.loop(0, RPC, step=L)
        def _(t):
            pid_v[pl.ds(nwin + t, L)] = jnp.zeros((L,), jnp.int32)
            dst_v[pl.ds(nwin + t, L)] = trash + t

        @pl.loop(0, cap // L)
        def _(i):
            dst2_v[i >> 3, pl.ds((i & 7) * L, L)] = dst_v[pl.ds(i * L, L)]

        def move(k, carry):
            gat = pltpu.async_copy(
                feat_hbm.at[pid_v.at[pl.ds(k * RPC, RPC)]], rows_v, sem)
            gat.wait()
            sc = pltpu.async_copy(rows_v, t_hbm.at[dst2_v.at[k]], sem)
            sc.wait()
            return carry

        lax.fori_loop(0, (nwin + RPC - 1) // RPC, move, jnp.int32(0))

    return body(idx, feat_ext)


def _tc_transpose(t, src):
    blk = 4096
    yb = blk // NX

    def body(t_ref, s_ref, o_ref):
        x = t_ref[...]
        s = s_ref[0, 0, :]
        for y in range(yb):
            m = (s[y * NX:(y + 1) * NX] >= 0)[None, :]
            o_ref[:, y, :] = jnp.where(m, x[y * NX:(y + 1) * NX, :C].T, 0.0)

    return pl.pallas_call(
        body,
        grid=(NSLOT // blk,),
        in_specs=[
            pl.BlockSpec((blk, 128), lambda i: (i, 0)),
            pl.BlockSpec((1, 1, blk), lambda i: (i, 0, 0)),
        ],
        out_specs=pl.BlockSpec((C, yb, NX), lambda i: (0, i, 0)),
        out_shape=jax.ShapeDtypeStruct((C, NY, NX), jnp.float32),
    )(t, src.reshape(NSLOT // blk, 1, blk))


def kernel(pillar_features, coords):
    idx = (coords[:, 0] + coords[:, 1] * NX + coords[:, 2]).astype(jnp.int32)
    feat_ext = _tc_build_table(pillar_features.T)
    t, src = _sc_scatter(idx, feat_ext)
    out = _tc_transpose(t, src)
    return out.reshape(1, C * NZ, NY, NX)


# speedup vs baseline: 2.2458x; 1.0835x over previous
import dataclasses
import functools

import jax
import jax.numpy as jnp
from jax import lax
from jax.experimental import pallas as pl
from jax.experimental.pallas import tpu as pltpu
from jax.experimental.pallas import tpu_sc as plsc

NX = 512
NY = 512
NZ = 1
C = 64
P = 32768
NSLOT = NX * NY * NZ
NCORES = 2
NSUB = 16
NW = NCORES * NSUB
SLOTS_PER_W = NSLOT // NW
L = 16
RPC = 128
NTRASH = NW * RPC
SENT = 0x7FFFFFFF


def _tc_build_table(feat_t):
    blk = 2048

    def body(x_ref, o_ref):
        o_ref[:, :C] = x_ref[...].T
        o_ref[:, C:] = jnp.zeros((blk, 128 - C), jnp.float32)

    return pl.pallas_call(
        body,
        grid=(P // blk,),
        in_specs=[pl.BlockSpec((C, blk), lambda i: (0, i))],
        out_specs=pl.BlockSpec((blk, 128), lambda i: (i, 0)),
        out_shape=jax.ShapeDtypeStruct((P, 128), jnp.float32),
    )(feat_t)


def _sc_scatter(idx, feat_ext):
    mesh = plsc.VectorSubcoreMesh(core_axis_name="c", subcore_axis_name="s")
    cp = pltpu.CompilerParams()
    if "needs_layout_passes" in pltpu.CompilerParams.__dataclass_fields__:
        cp = dataclasses.replace(cp, needs_layout_passes=False)

    nvec = SLOTS_PER_W // L
    cap = SLOTS_PER_W + RPC

    @functools.partial(
        pl.kernel,
        mesh=mesh,
        compiler_params=cp,
        out_type=(
            jax.ShapeDtypeStruct((NSLOT + NTRASH, 128), jnp.float32),
            jax.ShapeDtypeStruct((NSLOT,), jnp.int32),
        ),
        scratch_types=[
            pltpu.VMEM((P,), jnp.int32),
            pltpu.VMEM((SLOTS_PER_W,), jnp.int32),
            pltpu.VMEM((cap,), jnp.int32),
            pltpu.VMEM((cap,), jnp.int32),
            pltpu.VMEM((cap // RPC, RPC), jnp.int32),
            pltpu.VMEM((RPC, 128), jnp.float32),
            pltpu.SemaphoreType.DMA,
        ],
    )
    def body(idx_hbm, feat_hbm, t_hbm, src_hbm, idx_v, src_v,
             pid_v, dst_v, dst2_v, rows_v, sem):
        wid = lax.axis_index("c") * NSUB + lax.axis_index("s")
        base = wid * SLOTS_PER_W
        lanes = lax.iota(jnp.int32, L)

        pltpu.sync_copy(idx_hbm, idx_v)

        @pl.loop(0, SLOTS_PER_W, step=L)
        def _(j):
            src_v[pl.ds(j, L)] = jnp.full((L,), -1, jnp.int32)

        @pl.loop(0, P, step=4 * L)
        def _(p0):
            for u in range(4):
                v_idx = idx_v[pl.ds(p0 + u * L, L)]
                local = v_idx - base
                inr = local.astype(jnp.uint32) < jnp.uint32(SLOTS_PER_W)
                _, last = plsc.scan_count(local, mask=inr)
                plsc.store_scatter(src_v, [local], p0 + u * L + lanes,
                                   mask=last & inr)

        pltpu.sync_copy(src_v, src_hbm.at[pl.ds(base, SLOTS_PER_W)])

        def compact(k, off):
            v = src_v[pl.ds(k * L, L)]
            m = v >= 0
            plsc.store_compressed(pid_v.at[pl.ds(off, L)], v, mask=m)
            plsc.store_compressed(dst_v.at[pl.ds(off, L)],
                                  base + k * L + lanes, mask=m)
            return off + plsc.all_reduce_population_count(m)[0]

        nwin = lax.fori_loop(0, nvec, compact, jnp.int32(0))

        trash = NSLOT + wid * RPC + lanes

        @pl.loop(0, RPC, step=L)
        def _(t):
            pid_v[pl.ds(nwin + t, L)] = jnp.zeros((L,), jnp.int32)
            dst_v[pl.ds(nwin + t, L)] = trash + t

        @pl.loop(0, cap // L)
        def _(i):
            dst2_v[i >> 3, pl.ds((i & 7) * L, L)] = dst_v[pl.ds(i * L, L)]

        def move(k, carry):
            gat = pltpu.async_copy(
                feat_hbm.at[pid_v.at[pl.ds(k * RPC, RPC)]], rows_v, sem)
            gat.wait()
            sc = pltpu.async_copy(rows_v, t_hbm.at[dst2_v.at[k]], sem)
            sc.wait()
            return carry

        lax.fori_loop(0, (nwin + RPC - 1) // RPC, move, jnp.int32(0))

    return body(idx, feat_ext)


def _tc_transpose(t, src):
    blk = 8192
    yb = blk // NX

    def body(t_ref, s_ref, o_ref):
        x = t_ref[...]
        s = s_ref[0, 0, :]
        for y in range(yb):
            m = (s[y * NX:(y + 1) * NX] >= 0)[None, :]
            o_ref[:, y, :] = jnp.where(m, x[y * NX:(y + 1) * NX, :C].T, 0.0)

    return pl.pallas_call(
        body,
        grid=(NSLOT // blk,),
        in_specs=[
            pl.BlockSpec((blk, 128), lambda i: (i, 0)),
            pl.BlockSpec((1, 1, blk), lambda i: (i, 0, 0)),
        ],
        out_specs=pl.BlockSpec((C, yb, NX), lambda i: (0, i, 0)),
        out_shape=jax.ShapeDtypeStruct((C, NY, NX), jnp.float32),
    )(t, src.reshape(NSLOT // blk, 1, blk))


def kernel(pillar_features, coords):
    idx = (coords[:, 0] + coords[:, 1] * NX + coords[:, 2]).astype(jnp.int32)
    feat_ext = _tc_build_table(pillar_features.T)
    t, src = _sc_scatter(idx, feat_ext)
    out = _tc_transpose(t, src)
    return out.reshape(1, C * NZ, NY, NX)
